# Initial kernel scaffold; baseline (speedup 1.0000x reference)
import jax
import jax.numpy as jnp
from jax.experimental import pallas as pl


def kernel(x, neighbors, kernel):
    raise NotImplementedError("write your pallas kernel here")


# speedup vs baseline: 13.2999x; 13.2999x over previous
import functools

import jax
import jax.numpy as jnp
from jax import lax
from jax.experimental import pallas as pl
from jax.experimental.pallas import tpu as pltpu
from jax.experimental.pallas import tpu_sc as plsc

DEG = 16
D_OUT = 256
LANES = 16
N_WORKERS = 32
NODES_PER_W = 320
N_PAD = N_WORKERS * NODES_PER_W
CHUNK = 8
N_CHUNKS = NODES_PER_W // CHUNK


def _matmul_body(x_ref, w_ref, o_ref):
    o_ref[...] = jnp.dot(x_ref[...], w_ref[...],
                         preferred_element_type=jnp.float32)


def _matmul(x, w):
    n, d_in = x.shape
    d_out = w.shape[1]
    blk = 1000
    return pl.pallas_call(
        _matmul_body,
        grid=(n // blk,),
        in_specs=[
            pl.BlockSpec((blk, d_in), lambda i: (i, 0)),
            pl.BlockSpec((d_in, d_out), lambda i: (0, 0)),
        ],
        out_specs=pl.BlockSpec((blk, d_out), lambda i: (i, 0)),
        out_shape=jax.ShapeDtypeStruct((n, d_out), jnp.float32),
    )(x, w)


_SORT8 = [(0, 1), (2, 3), (4, 5), (6, 7),
          (0, 2), (1, 3), (1, 2),
          (4, 6), (5, 7), (5, 6),
          (0, 4), (1, 5), (2, 6), (3, 7),
          (2, 4), (3, 5),
          (1, 2), (3, 4), (5, 6)]


def _sort8(v):
    v = list(v)
    for i, j in _SORT8:
        lo = jnp.minimum(v[i], v[j])
        hi = jnp.maximum(v[i], v[j])
        v[i], v[j] = lo, hi
    return v


def _median16(vs):
    a = _sort8(vs[:8])
    b = _sort8(vs[8:])
    m = [jnp.minimum(a[i], b[7 - i]) for i in range(8)]
    while len(m) > 1:
        nxt = [jnp.maximum(m[2 * i], m[2 * i + 1]) for i in range(len(m) // 2)]
        if len(m) % 2:
            nxt.append(m[-1])
        m = nxt
    return m[0]


@functools.partial(
    pl.kernel,
    out_type=jax.ShapeDtypeStruct((N_PAD, D_OUT), jnp.float32),
    mesh=plsc.VectorSubcoreMesh(core_axis_name="c", subcore_axis_name="s"),
    scratch_types=[
        pltpu.VMEM((CHUNK * DEG,), jnp.int32),
        pltpu.VMEM((CHUNK * DEG, D_OUT), jnp.float32),
        pltpu.VMEM((CHUNK, D_OUT), jnp.float32),
        pltpu.SemaphoreType.DMA,
    ],
)
def _sc_gather_median(h_hbm, nbr_hbm, out_hbm, idx_v, rows_v, out_v, sem):
    wid = lax.axis_index("s") * 2 + lax.axis_index("c")

    def chunk_body(t, carry):
        base = wid * NODES_PER_W + t * CHUNK
        pltpu.sync_copy(nbr_hbm.at[pl.ds(base * DEG, CHUNK * DEG)], idx_v)
        pltpu.async_copy(h_hbm.at[idx_v], rows_v, sem).wait()

        def node_body(j, carry2):
            def group_body(g, carry3):
                c0 = g * LANES
                vs = [rows_v[j * DEG + i, pl.ds(c0, LANES)]
                      for i in range(DEG)]
                out_v[j, pl.ds(c0, LANES)] = _median16(vs)
                return carry3

            lax.fori_loop(0, D_OUT // LANES, group_body, 0, unroll=False)
            return carry2

        lax.fori_loop(0, CHUNK, node_body, 0, unroll=False)
        pltpu.sync_copy(out_v, out_hbm.at[pl.ds(base, CHUNK)])
        return carry

    lax.fori_loop(0, N_CHUNKS, chunk_body, 0, unroll=False)


def kernel(x, neighbors, kernel):
    n = x.shape[0]
    h = _matmul(x, kernel)
    nbr = neighbors.astype(jnp.int32)
    nbr_flat = jnp.pad(nbr, ((0, N_PAD - n), (0, 0))).reshape(-1)
    out = _sc_gather_median(h, nbr_flat)
    return out[:n]
